# baseline (speedup 1.0000x reference)
import functools

import jax
import jax.numpy as jnp
from jax import lax
from jax.experimental import pallas as pl
from jax.experimental.pallas import tpu as pltpu
from jax.experimental.pallas import tpu_sc as plsc

BATCH = 16384
D = 16

_info = plsc.get_sparse_core_info()
NC = _info.num_cores
NS = _info.num_subcores
L = _info.num_lanes
NW = NC * NS
BPW = BATCH // NW
CHUNK = 128
NCHUNK = BPW // CHUNK
GROUPS = BPW // L

_mesh = plsc.VectorSubcoreMesh(core_axis_name="c", subcore_axis_name="s")


@functools.partial(
    pl.kernel,
    mesh=_mesh,
    out_type=jax.ShapeDtypeStruct((BATCH,), jnp.float32),
    scratch_types=[
        pltpu.VMEM((NCHUNK, CHUNK), jnp.int32),
        pltpu.VMEM((NCHUNK, CHUNK), jnp.int32),
        pltpu.VMEM((BPW, D), jnp.float32),
        pltpu.VMEM((BPW, D), jnp.float32),
        pltpu.VMEM((BPW,), jnp.float32),
        pltpu.SemaphoreType.DMA,
        pltpu.SemaphoreType.DMA,
    ],
    compiler_params=pltpu.CompilerParams(
        needs_layout_passes=False, use_tc_tiling_on_sc=False
    ),
)
def _pmf_sc(uidx_hbm, iidx_hbm, wu_hbm, wi_hbm, out_hbm,
            uidx_v, iidx_v, urows_v, irows_v, out_v, usem, isem):
    wid = lax.axis_index("s") * NC + lax.axis_index("c")
    base_row = wid * NCHUNK

    pltpu.sync_copy(uidx_hbm.at[pl.ds(base_row, NCHUNK)], uidx_v)
    pltpu.sync_copy(iidx_hbm.at[pl.ds(base_row, NCHUNK)], iidx_v)

    ucopies = []
    icopies = []
    for c in range(NCHUNK):
        dst_u = urows_v.at[pl.ds(c * CHUNK, CHUNK), :]
        dst_i = irows_v.at[pl.ds(c * CHUNK, CHUNK), :]
        ucopies.append(pltpu.async_copy(wu_hbm.at[uidx_v.at[c]], dst_u, usem))
        icopies.append(pltpu.async_copy(wi_hbm.at[iidx_v.at[c]], dst_i, isem))
    for cp in ucopies:
        cp.wait()
    for cp in icopies:
        cp.wait()

    lane = lax.iota(jnp.int32, L)

    def group_body(g, _):
        row0 = g * L
        row_idx = row0 + lane
        acc = jnp.zeros((L,), jnp.float32)
        for d in range(D):
            col_idx = jnp.full((L,), d, jnp.int32)
            uc = plsc.load_gather(urows_v, [row_idx, col_idx])
            ic = plsc.load_gather(irows_v, [row_idx, col_idx])
            acc = acc + uc * ic
        out_v[pl.ds(row0, L)] = acc
        return 0

    lax.fori_loop(0, GROUPS, group_body, 0)

    pltpu.sync_copy(out_v, out_hbm.at[pl.ds(wid * BPW, BPW)])


def kernel(uesr_indices, item_indices, w_user, w_item):
    uidx = uesr_indices.astype(jnp.int32).reshape(NW * NCHUNK, CHUNK)
    iidx = item_indices.astype(jnp.int32).reshape(NW * NCHUNK, CHUNK)
    return _pmf_sc(uidx, iidx, w_user, w_item)
